# baseline (speedup 1.0000x reference)
import functools

import jax
import jax.numpy as jnp
from jax import lax
from jax.experimental import pallas as pl
from jax.experimental.pallas import tpu as pltpu
from jax.experimental.pallas import tpu_sc as plsc

N = 10000
E = 160000
D_IN = 256
D_HID = 256
D_OUT = 40
D_OUT_PAD = 64

NC = 2
NS = 16
K = 80
KD = 40
CB = 64
NBUF = 5
DGRP = 5

EPT = E // NS
CH = EPT // K
EPT_D = E // (NC * NS)
CH_D = EPT_D // KD
FR = 632
TAIL = N - (NS - 1) * FR
BN = 1000


def _mesh():
    return plsc.VectorSubcoreMesh(
        core_axis_name="c", subcore_axis_name="s",
        num_cores=NC, num_subcores=NS)


def _sc_degree(dst_d):
    ones_h = jnp.ones((KD,), jnp.float32)
    zeros_h = jnp.zeros((N,), jnp.float32)

    @functools.partial(
        pl.kernel,
        out_type=jax.ShapeDtypeStruct((NC, N), jnp.float32),
        mesh=_mesh(),
        scratch_types=[
            pltpu.VMEM((CH_D, KD), jnp.int32),
            pltpu.VMEM((KD,), jnp.float32),
            pltpu.VMEM_SHARED((N,), jnp.float32),
            pltpu.SemaphoreType.DMA,
        ],
    )
    def kern(dst_h, ones_hbm, z_hbm, deg_h, didx, ones_v, deg_sh, sem):
        c = lax.axis_index("c")
        s = lax.axis_index("s")
        t = c * NS + s
        pltpu.sync_copy(dst_h.at[t], didx)
        pltpu.sync_copy(ones_hbm, ones_v)

        @pl.when(s == 0)
        def _zero():
            pltpu.sync_copy(z_hbm, deg_sh)

        plsc.subcore_barrier()

        @pl.loop(0, CH_D, step=DGRP)
        def _body(j0):
            descs = [
                pltpu.async_copy(ones_v, deg_sh.at[didx.at[j0 + g]], sem,
                                 add=True)
                for g in range(DGRP)
            ]
            for desc in descs:
                desc.wait()

        plsc.subcore_barrier()

        @pl.when(s == 0)
        def _wb():
            pltpu.sync_copy(deg_sh, deg_h.at[c])

    return kern(dst_d, ones_h, zeros_h)


@functools.cache
def _sc_spmm2_kern():
    @functools.partial(
        pl.kernel,
        out_type=jax.ShapeDtypeStruct((4 * N, CB), jnp.float32),
        mesh=_mesh(),
        compiler_params=pltpu.CompilerParams(use_tc_tiling_on_sc=False),
        scratch_types=[
            pltpu.VMEM((CH, K), jnp.int32),
            pltpu.VMEM((CH, K), jnp.int32),
            pltpu.VMEM((2, NBUF, K, CB), jnp.float32),
            pltpu.VMEM_SHARED((N, CB), jnp.float32),
            pltpu.SemaphoreType.DMA,
            pltpu.SemaphoreType.DMA,
        ],
    )
    def kern(m_h, src_h, dst_h, z_h, agg_h, sidx, didx, rows, acc_sh,
             gsem, ssem):
        c = lax.axis_index("c")
        s = lax.axis_index("s")
        pltpu.sync_copy(dst_h.at[s], didx)

        for p in range(2):
            q = 2 * p + c
            pltpu.sync_copy(src_h.at[p * NC * NS + c * NS + s], sidx)

            @pl.when(s < NS - 1)
            def _zero():
                pltpu.sync_copy(z_h, acc_sh.at[pl.ds(s * FR, FR)])

            @pl.when(s == NS - 1)
            def _zero_t():
                pltpu.sync_copy(z_h.at[pl.ds(0, TAIL)],
                                acc_sh.at[pl.ds((NS - 1) * FR, TAIL)])

            plsc.subcore_barrier()

            @pl.loop(0, CH, step=NBUF)
            def _body(j0):
                half = (j0 // NBUF) % 2

                @pl.when(j0 >= 2 * NBUF)
                def _drain_old():
                    for b in range(NBUF):
                        pltpu.make_async_copy(m_h.at[sidx.at[j0]],
                                              rows.at[0, b], ssem).wait()

                gets = [
                    pltpu.async_copy(m_h.at[sidx.at[j0 + b]],
                                     rows.at[half, b], gsem)
                    for b in range(NBUF)
                ]
                for b in range(NBUF):
                    gets[b].wait()
                    pltpu.async_copy(rows.at[half, b],
                                     acc_sh.at[didx.at[j0 + b]], ssem,
                                     add=True)

            for i in range(2 * NBUF):
                pltpu.make_async_copy(m_h.at[sidx.at[0]],
                                      rows.at[0, i % NBUF], ssem).wait()

            plsc.subcore_barrier()

            @pl.when(s < NS - 1)
            def _wb():
                pltpu.sync_copy(acc_sh.at[pl.ds(s * FR, FR)],
                                agg_h.at[pl.ds(q * N + s * FR, FR)])

            @pl.when(s == NS - 1)
            def _wb_t():
                pltpu.sync_copy(acc_sh.at[pl.ds((NS - 1) * FR, TAIL)],
                                agg_h.at[pl.ds(q * N + (NS - 1) * FR, TAIL)])

    return kern


def _sc_spmm2(mflat, srcb2, dst):
    zrows = jnp.zeros((FR, CB), jnp.float32)
    return _sc_spmm2_kern()(mflat, srcb2, dst, zrows)


@functools.cache
def _sc_spmm_l3_kern():
    @functools.partial(
        pl.kernel,
        out_type=jax.ShapeDtypeStruct((NC * N, CB), jnp.float32),
        mesh=_mesh(),
        compiler_params=pltpu.CompilerParams(use_tc_tiling_on_sc=False),
        scratch_types=[
            pltpu.VMEM((CH_D, KD), jnp.int32),
            pltpu.VMEM((CH_D, KD), jnp.int32),
            pltpu.VMEM((2, NBUF, KD, CB), jnp.float32),
            pltpu.VMEM_SHARED((N, CB), jnp.float32),
            pltpu.SemaphoreType.DMA,
            pltpu.SemaphoreType.DMA,
        ],
    )
    def kern(m_h, src_h, dst_h, z_h, agg_h, sidx, didx, rows, acc_sh,
             gsem, ssem):
        c = lax.axis_index("c")
        s = lax.axis_index("s")
        t = c * NS + s
        pltpu.sync_copy(src_h.at[t], sidx)
        pltpu.sync_copy(dst_h.at[t], didx)

        @pl.when(s < NS - 1)
        def _zero():
            pltpu.sync_copy(z_h, acc_sh.at[pl.ds(s * FR, FR)])

        @pl.when(s == NS - 1)
        def _zero_t():
            pltpu.sync_copy(z_h.at[pl.ds(0, TAIL)],
                            acc_sh.at[pl.ds((NS - 1) * FR, TAIL)])

        plsc.subcore_barrier()

        @pl.loop(0, CH_D, step=NBUF)
        def _body(j0):
            half = (j0 // NBUF) % 2

            @pl.when(j0 >= 2 * NBUF)
            def _drain_old():
                for b in range(NBUF):
                    pltpu.make_async_copy(m_h.at[sidx.at[j0]],
                                          rows.at[0, b], ssem).wait()

            gets = [
                pltpu.async_copy(m_h.at[sidx.at[j0 + b]],
                                 rows.at[half, b], gsem)
                for b in range(NBUF)
            ]
            for b in range(NBUF):
                gets[b].wait()
                pltpu.async_copy(rows.at[half, b],
                                 acc_sh.at[didx.at[j0 + b]], ssem, add=True)

        for i in range(2 * NBUF):
            pltpu.make_async_copy(m_h.at[sidx.at[0]],
                                  rows.at[0, i % NBUF], ssem).wait()

        plsc.subcore_barrier()

        @pl.when(s < NS - 1)
        def _wb():
            pltpu.sync_copy(acc_sh.at[pl.ds(s * FR, FR)],
                            agg_h.at[pl.ds(c * N + s * FR, FR)])

        @pl.when(s == NS - 1)
        def _wb_t():
            pltpu.sync_copy(acc_sh.at[pl.ds((NS - 1) * FR, TAIL)],
                            agg_h.at[pl.ds(c * N + (NS - 1) * FR, TAIL)])

    return kern


def _r_of(deg_ref):
    d = deg_ref[0] + deg_ref[1]
    return lax.rsqrt(jnp.maximum(d, 1.0))


def _tc_matmul1(x, W1):
    def body(x_ref, w_ref, out_ref):
        z = jnp.dot(x_ref[...], w_ref[...],
                    preferred_element_type=jnp.float32)
        for q in range(4):
            out_ref[q] = z[:, q * CB:(q + 1) * CB]

    return pl.pallas_call(
        body,
        grid=(N // BN,),
        in_specs=[
            pl.BlockSpec((BN, D_IN), lambda i: (i, 0)),
            pl.BlockSpec((D_IN, D_HID), lambda i: (0, 0)),
        ],
        out_specs=pl.BlockSpec((4, BN, CB), lambda i: (0, i, 0)),
        out_shape=jax.ShapeDtypeStruct((4, N, CB), jnp.float32),
    )(x, W1)


def _tc_scale1(z, degp):
    def body(z_ref, deg_ref, out_ref):
        r = _r_of(deg_ref)
        for q in range(4):
            out_ref[q] = z_ref[q] * r

    return pl.pallas_call(
        body,
        grid=(N // BN,),
        in_specs=[
            pl.BlockSpec((4, BN, CB), lambda i: (0, i, 0)),
            pl.BlockSpec((NC, BN, 1), lambda i: (0, i, 0)),
        ],
        out_specs=pl.BlockSpec((4, BN, CB), lambda i: (0, i, 0)),
        out_shape=jax.ShapeDtypeStruct((4, N, CB), jnp.float32),
    )(z, degp)


def _tc_stage_mid(agg, degp, b, W, nq):
    d_out = nq * CB

    def body(a_ref, deg_ref, b_ref, w_ref, out_ref):
        r = _r_of(deg_ref)
        h = jnp.concatenate(
            [a_ref[0], a_ref[1], a_ref[2], a_ref[3]], axis=1)
        h = jnp.maximum(h * r + b_ref[...], 0.0)
        z = jnp.dot(h, w_ref[...], preferred_element_type=jnp.float32) * r
        for q in range(nq):
            out_ref[q] = z[:, q * CB:(q + 1) * CB]

    return pl.pallas_call(
        body,
        grid=(N // BN,),
        in_specs=[
            pl.BlockSpec((4, BN, CB), lambda i: (0, i, 0)),
            pl.BlockSpec((NC, BN, 1), lambda i: (0, i, 0)),
            pl.BlockSpec((1, D_HID), lambda i: (0, 0)),
            pl.BlockSpec((D_HID, d_out), lambda i: (0, 0)),
        ],
        out_specs=pl.BlockSpec((nq, BN, CB), lambda i: (0, i, 0)),
        out_shape=jax.ShapeDtypeStruct((nq, N, CB), jnp.float32),
    )(agg, degp, b, W)


def _tc_stage_final(agg3, degp, b3p):
    def body(agg_ref, deg_ref, b_ref, out_ref):
        r = _r_of(deg_ref)
        o = agg_ref[0] + agg_ref[1]
        o = o * r + b_ref[...]
        col = lax.broadcasted_iota(jnp.int32, (BN, D_OUT_PAD), 1)
        valid = col < D_OUT
        om = jnp.where(valid, o, jnp.float32(-1e30))
        m = jnp.max(om, axis=1, keepdims=True)
        e = jnp.where(valid, jnp.exp(om - m), 0.0)
        lse = jnp.log(jnp.sum(e, axis=1, keepdims=True)) + m
        res = o - lse
        out_ref[...] = res[:, :D_OUT]

    return pl.pallas_call(
        body,
        grid=(N // BN,),
        in_specs=[
            pl.BlockSpec((NC, BN, CB), lambda i: (0, i, 0)),
            pl.BlockSpec((NC, BN, 1), lambda i: (0, i, 0)),
            pl.BlockSpec((1, D_OUT_PAD), lambda i: (0, 0)),
        ],
        out_specs=pl.BlockSpec((BN, D_OUT), lambda i: (i, 0)),
        out_shape=jax.ShapeDtypeStruct((N, D_OUT), jnp.float32),
    )(agg3, degp, b3p)


def kernel(x, edge_index, W1, b1, W2, b2, W3, b3):
    src = edge_index[0]
    dst = edge_index[1]

    src3 = src.reshape(NC * NS, CH_D, KD)
    dst3 = dst.reshape(NC * NS, CH_D, KD)
    dstp = dst.reshape(NS, CH, K)
    offq = (jnp.arange(4, dtype=jnp.int32) * N).reshape(2, NC, 1, 1, 1)
    srcb2 = (src.reshape(1, 1, NS, CH, K) + offq).reshape(
        2 * NC * NS, CH, K)

    W3p = jnp.pad(W3, ((0, 0), (0, D_OUT_PAD - D_OUT)))
    b3p = jnp.pad(b3, (0, D_OUT_PAD - D_OUT)).reshape(1, D_OUT_PAD)

    z1 = _tc_matmul1(x, W1)
    degp = _sc_degree(dst3).reshape(NC, N, 1)

    m1 = _tc_scale1(z1, degp).reshape(4 * N, CB)
    a1 = _sc_spmm2(m1, srcb2, dstp).reshape(4, N, CB)
    m2 = _tc_stage_mid(a1, degp, b1.reshape(1, D_HID), W2,
                       nq=4).reshape(4 * N, CB)
    a2 = _sc_spmm2(m2, srcb2, dstp).reshape(4, N, CB)
    m3 = _tc_stage_mid(a2, degp, b2.reshape(1, D_HID), W3p,
                       nq=1).reshape(N, CB)
    a3 = _sc_spmm_l3_kern()(m3, src3, dst3,
                            jnp.zeros((FR, CB), jnp.float32)).reshape(NC, N, CB)
    return _tc_stage_final(a3, degp, b3p)


# speedup vs baseline: 1.0321x; 1.0321x over previous
import functools

import jax
import jax.numpy as jnp
from jax import lax
from jax.experimental import pallas as pl
from jax.experimental.pallas import tpu as pltpu
from jax.experimental.pallas import tpu_sc as plsc

N = 10000
E = 160000
D_IN = 256
D_HID = 256
D_OUT = 40
D_OUT_PAD = 64

NC = 2
NS = 16
K = 80
KD = 40
CB = 64
NBUF = 5
DGRP = 25

EPT = E // NS
CH = EPT // K
EPT_D = E // (NC * NS)
CH_D = EPT_D // KD
FR = 632
TAIL = N - (NS - 1) * FR
BN = 1000


def _mesh():
    return plsc.VectorSubcoreMesh(
        core_axis_name="c", subcore_axis_name="s",
        num_cores=NC, num_subcores=NS)


def _sc_degree(dst_d):
    ones_h = jnp.ones((KD,), jnp.float32)
    zeros_h = jnp.zeros((N,), jnp.float32)

    @functools.partial(
        pl.kernel,
        out_type=jax.ShapeDtypeStruct((NC, N), jnp.float32),
        mesh=_mesh(),
        compiler_params=pltpu.CompilerParams(skip_device_barrier=True),
        scratch_types=[
            pltpu.VMEM((CH_D, KD), jnp.int32),
            pltpu.VMEM((KD,), jnp.float32),
            pltpu.VMEM_SHARED((N,), jnp.float32),
            pltpu.SemaphoreType.DMA,
        ],
    )
    def kern(dst_h, ones_hbm, z_hbm, deg_h, didx, ones_v, deg_sh, sem):
        c = lax.axis_index("c")
        s = lax.axis_index("s")
        t = c * NS + s
        pltpu.sync_copy(dst_h.at[t], didx)
        pltpu.sync_copy(ones_hbm, ones_v)

        @pl.when(s == 0)
        def _zero():
            pltpu.sync_copy(z_hbm, deg_sh)

        plsc.subcore_barrier()

        @pl.loop(0, CH_D, step=DGRP)
        def _body(j0):
            descs = [
                pltpu.async_copy(ones_v, deg_sh.at[didx.at[j0 + g]], sem,
                                 add=True)
                for g in range(DGRP)
            ]
            for desc in descs:
                desc.wait()

        plsc.subcore_barrier()

        @pl.when(s == 0)
        def _wb():
            pltpu.sync_copy(deg_sh, deg_h.at[c])

    return kern(dst_d, ones_h, zeros_h)


@functools.cache
def _sc_spmm2_kern():
    @functools.partial(
        pl.kernel,
        out_type=jax.ShapeDtypeStruct((4 * N, CB), jnp.float32),
        mesh=_mesh(),
        compiler_params=pltpu.CompilerParams(use_tc_tiling_on_sc=False,
                                             skip_device_barrier=True),
        scratch_types=[
            pltpu.VMEM((CH, K), jnp.int32),
            pltpu.VMEM((CH, K), jnp.int32),
            pltpu.VMEM((2, NBUF, K, CB), jnp.float32),
            pltpu.VMEM_SHARED((N, CB), jnp.float32),
            pltpu.SemaphoreType.DMA,
            pltpu.SemaphoreType.DMA,
        ],
    )
    def kern(m_h, src_h, dst_h, z_h, agg_h, sidx, didx, rows, acc_sh,
             gsem, ssem):
        c = lax.axis_index("c")
        s = lax.axis_index("s")
        pltpu.sync_copy(dst_h.at[s], didx)

        for p in range(2):
            q = 2 * p + c
            pltpu.sync_copy(src_h.at[p * NC * NS + c * NS + s], sidx)

            @pl.when(s < NS - 1)
            def _zero():
                pltpu.sync_copy(z_h, acc_sh.at[pl.ds(s * FR, FR)])

            @pl.when(s == NS - 1)
            def _zero_t():
                pltpu.sync_copy(z_h.at[pl.ds(0, TAIL)],
                                acc_sh.at[pl.ds((NS - 1) * FR, TAIL)])

            plsc.subcore_barrier()

            @pl.loop(0, CH, step=NBUF)
            def _body(j0):
                half = (j0 // NBUF) % 2

                @pl.when(j0 >= 2 * NBUF)
                def _drain_old():
                    for b in range(NBUF):
                        pltpu.make_async_copy(m_h.at[sidx.at[j0]],
                                              rows.at[0, b], ssem).wait()

                gets = [
                    pltpu.async_copy(m_h.at[sidx.at[j0 + b]],
                                     rows.at[half, b], gsem)
                    for b in range(NBUF)
                ]
                for b in range(NBUF):
                    gets[b].wait()
                    pltpu.async_copy(rows.at[half, b],
                                     acc_sh.at[didx.at[j0 + b]], ssem,
                                     add=True)

            for i in range(2 * NBUF):
                pltpu.make_async_copy(m_h.at[sidx.at[0]],
                                      rows.at[0, i % NBUF], ssem).wait()

            plsc.subcore_barrier()

            @pl.when(s < NS - 1)
            def _wb():
                pltpu.sync_copy(acc_sh.at[pl.ds(s * FR, FR)],
                                agg_h.at[pl.ds(q * N + s * FR, FR)])

            @pl.when(s == NS - 1)
            def _wb_t():
                pltpu.sync_copy(acc_sh.at[pl.ds((NS - 1) * FR, TAIL)],
                                agg_h.at[pl.ds(q * N + (NS - 1) * FR, TAIL)])

    return kern


def _sc_spmm2(mflat, srcb2, dst):
    zrows = jnp.zeros((FR, CB), jnp.float32)
    return _sc_spmm2_kern()(mflat, srcb2, dst, zrows)


@functools.cache
def _sc_spmm_l3_kern():
    @functools.partial(
        pl.kernel,
        out_type=jax.ShapeDtypeStruct((NC * N, CB), jnp.float32),
        mesh=_mesh(),
        compiler_params=pltpu.CompilerParams(use_tc_tiling_on_sc=False,
                                             skip_device_barrier=True),
        scratch_types=[
            pltpu.VMEM((CH_D, KD), jnp.int32),
            pltpu.VMEM((CH_D, KD), jnp.int32),
            pltpu.VMEM((2, NBUF, KD, CB), jnp.float32),
            pltpu.VMEM_SHARED((N, CB), jnp.float32),
            pltpu.SemaphoreType.DMA,
            pltpu.SemaphoreType.DMA,
        ],
    )
    def kern(m_h, src_h, dst_h, z_h, agg_h, sidx, didx, rows, acc_sh,
             gsem, ssem):
        c = lax.axis_index("c")
        s = lax.axis_index("s")
        t = c * NS + s
        pltpu.sync_copy(src_h.at[t], sidx)
        pltpu.sync_copy(dst_h.at[t], didx)

        @pl.when(s < NS - 1)
        def _zero():
            pltpu.sync_copy(z_h, acc_sh.at[pl.ds(s * FR, FR)])

        @pl.when(s == NS - 1)
        def _zero_t():
            pltpu.sync_copy(z_h.at[pl.ds(0, TAIL)],
                            acc_sh.at[pl.ds((NS - 1) * FR, TAIL)])

        plsc.subcore_barrier()

        @pl.loop(0, CH_D, step=NBUF)
        def _body(j0):
            half = (j0 // NBUF) % 2

            @pl.when(j0 >= 2 * NBUF)
            def _drain_old():
                for b in range(NBUF):
                    pltpu.make_async_copy(m_h.at[sidx.at[j0]],
                                          rows.at[0, b], ssem).wait()

            gets = [
                pltpu.async_copy(m_h.at[sidx.at[j0 + b]],
                                 rows.at[half, b], gsem)
                for b in range(NBUF)
            ]
            for b in range(NBUF):
                gets[b].wait()
                pltpu.async_copy(rows.at[half, b],
                                 acc_sh.at[didx.at[j0 + b]], ssem, add=True)

        for i in range(2 * NBUF):
            pltpu.make_async_copy(m_h.at[sidx.at[0]],
                                  rows.at[0, i % NBUF], ssem).wait()

        plsc.subcore_barrier()

        @pl.when(s < NS - 1)
        def _wb():
            pltpu.sync_copy(acc_sh.at[pl.ds(s * FR, FR)],
                            agg_h.at[pl.ds(c * N + s * FR, FR)])

        @pl.when(s == NS - 1)
        def _wb_t():
            pltpu.sync_copy(acc_sh.at[pl.ds((NS - 1) * FR, TAIL)],
                            agg_h.at[pl.ds(c * N + (NS - 1) * FR, TAIL)])

    return kern


def _r_of(deg_ref):
    d = deg_ref[0] + deg_ref[1]
    return lax.rsqrt(jnp.maximum(d, 1.0))


def _tc_stage1(x, W1, degp):
    def body(x_ref, w_ref, deg_ref, out_ref):
        r = _r_of(deg_ref)
        z = jnp.dot(x_ref[...], w_ref[...],
                    preferred_element_type=jnp.float32) * r
        for q in range(4):
            out_ref[q] = z[:, q * CB:(q + 1) * CB]

    return pl.pallas_call(
        body,
        grid=(N // BN,),
        in_specs=[
            pl.BlockSpec((BN, D_IN), lambda i: (i, 0)),
            pl.BlockSpec((D_IN, D_HID), lambda i: (0, 0)),
            pl.BlockSpec((NC, BN, 1), lambda i: (0, i, 0)),
        ],
        out_specs=pl.BlockSpec((4, BN, CB), lambda i: (0, i, 0)),
        out_shape=jax.ShapeDtypeStruct((4, N, CB), jnp.float32),
    )(x, W1, degp)


def _tc_stage_mid(agg, degp, b, W, nq):
    d_out = nq * CB

    def body(a_ref, deg_ref, b_ref, w_ref, out_ref):
        r = _r_of(deg_ref)
        h = jnp.concatenate(
            [a_ref[0], a_ref[1], a_ref[2], a_ref[3]], axis=1)
        h = jnp.maximum(h * r + b_ref[...], 0.0)
        z = jnp.dot(h, w_ref[...], preferred_element_type=jnp.float32) * r
        for q in range(nq):
            out_ref[q] = z[:, q * CB:(q + 1) * CB]

    return pl.pallas_call(
        body,
        grid=(N // BN,),
        in_specs=[
            pl.BlockSpec((4, BN, CB), lambda i: (0, i, 0)),
            pl.BlockSpec((NC, BN, 1), lambda i: (0, i, 0)),
            pl.BlockSpec((1, D_HID), lambda i: (0, 0)),
            pl.BlockSpec((D_HID, d_out), lambda i: (0, 0)),
        ],
        out_specs=pl.BlockSpec((nq, BN, CB), lambda i: (0, i, 0)),
        out_shape=jax.ShapeDtypeStruct((nq, N, CB), jnp.float32),
    )(agg, degp, b, W)


def _tc_stage_final(agg3, degp, b3p):
    def body(agg_ref, deg_ref, b_ref, out_ref):
        r = _r_of(deg_ref)
        o = agg_ref[0] + agg_ref[1]
        o = o * r + b_ref[...]
        col = lax.broadcasted_iota(jnp.int32, (BN, D_OUT_PAD), 1)
        valid = col < D_OUT
        om = jnp.where(valid, o, jnp.float32(-1e30))
        m = jnp.max(om, axis=1, keepdims=True)
        e = jnp.where(valid, jnp.exp(om - m), 0.0)
        lse = jnp.log(jnp.sum(e, axis=1, keepdims=True)) + m
        res = o - lse
        out_ref[...] = res[:, :D_OUT]

    return pl.pallas_call(
        body,
        grid=(N // BN,),
        in_specs=[
            pl.BlockSpec((NC, BN, CB), lambda i: (0, i, 0)),
            pl.BlockSpec((NC, BN, 1), lambda i: (0, i, 0)),
            pl.BlockSpec((1, D_OUT_PAD), lambda i: (0, 0)),
        ],
        out_specs=pl.BlockSpec((BN, D_OUT), lambda i: (i, 0)),
        out_shape=jax.ShapeDtypeStruct((N, D_OUT), jnp.float32),
    )(agg3, degp, b3p)


def kernel(x, edge_index, W1, b1, W2, b2, W3, b3):
    src = edge_index[0]
    dst = edge_index[1]

    src3 = src.reshape(NC * NS, CH_D, KD)
    dst3 = dst.reshape(NC * NS, CH_D, KD)
    dstp = dst.reshape(NS, CH, K)
    offq = (jnp.arange(4, dtype=jnp.int32) * N).reshape(2, NC, 1, 1, 1)
    srcb2 = (src.reshape(1, 1, NS, CH, K) + offq).reshape(
        2 * NC * NS, CH, K)

    W3p = jnp.pad(W3, ((0, 0), (0, D_OUT_PAD - D_OUT)))
    b3p = jnp.pad(b3, (0, D_OUT_PAD - D_OUT)).reshape(1, D_OUT_PAD)

    degp = _sc_degree(dst3).reshape(NC, N, 1)

    m1 = _tc_stage1(x, W1, degp).reshape(4 * N, CB)
    a1 = _sc_spmm2(m1, srcb2, dstp).reshape(4, N, CB)
    m2 = _tc_stage_mid(a1, degp, b1.reshape(1, D_HID), W2,
                       nq=4).reshape(4 * N, CB)
    a2 = _sc_spmm2(m2, srcb2, dstp).reshape(4, N, CB)
    m3 = _tc_stage_mid(a2, degp, b2.reshape(1, D_HID), W3p,
                       nq=1).reshape(N, CB)
    a3 = _sc_spmm_l3_kern()(m3, src3, dst3,
                            jnp.zeros((FR, CB), jnp.float32)).reshape(NC, N, CB)
    return _tc_stage_final(a3, degp, b3p)


# speedup vs baseline: 1.0323x; 1.0002x over previous
import functools

import jax
import jax.numpy as jnp
from jax import lax
from jax.experimental import pallas as pl
from jax.experimental.pallas import tpu as pltpu
from jax.experimental.pallas import tpu_sc as plsc

N = 10000
E = 160000
D_IN = 256
D_HID = 256
D_OUT = 40
D_OUT_PAD = 64

NC = 2
NS = 16
K = 80
KD = 40
CB = 64
NBUF = 5
DGRP = 25

EPT = E // NS
CH = EPT // K
EPT_D = E // (NC * NS)
CH_D = EPT_D // KD
FR = 632
TAIL = N - (NS - 1) * FR
BN = 1000


def _mesh():
    return plsc.VectorSubcoreMesh(
        core_axis_name="c", subcore_axis_name="s",
        num_cores=NC, num_subcores=NS)


def _sc_degree(dst_d):
    ones_h = jnp.ones((KD,), jnp.float32)
    zeros_h = jnp.zeros((N,), jnp.float32)

    @functools.partial(
        pl.kernel,
        out_type=jax.ShapeDtypeStruct((NC, N), jnp.float32),
        mesh=_mesh(),
        compiler_params=pltpu.CompilerParams(skip_device_barrier=True),
        scratch_types=[
            pltpu.VMEM((CH_D, KD), jnp.int32),
            pltpu.VMEM((KD,), jnp.float32),
            pltpu.VMEM_SHARED((N,), jnp.float32),
            pltpu.SemaphoreType.DMA,
        ],
    )
    def kern(dst_h, ones_hbm, z_hbm, deg_h, didx, ones_v, deg_sh, sem):
        c = lax.axis_index("c")
        s = lax.axis_index("s")
        t = c * NS + s
        pltpu.sync_copy(dst_h.at[t], didx)
        pltpu.sync_copy(ones_hbm, ones_v)

        @pl.when(s == 0)
        def _zero():
            pltpu.sync_copy(z_hbm, deg_sh)

        plsc.subcore_barrier()

        @pl.loop(0, CH_D, step=DGRP)
        def _body(j0):
            descs = [
                pltpu.async_copy(ones_v, deg_sh.at[didx.at[j0 + g]], sem,
                                 add=True)
                for g in range(DGRP)
            ]
            for desc in descs:
                desc.wait()

        plsc.subcore_barrier()

        @pl.when(s == 0)
        def _wb():
            pltpu.sync_copy(deg_sh, deg_h.at[c])

    return kern(dst_d, ones_h, zeros_h)


@functools.cache
def _sc_spmm2_kern():
    @functools.partial(
        pl.kernel,
        out_type=jax.ShapeDtypeStruct((4 * N, CB), jnp.float32),
        mesh=_mesh(),
        compiler_params=pltpu.CompilerParams(use_tc_tiling_on_sc=False,
                                             skip_device_barrier=True),
        scratch_types=[
            pltpu.VMEM((CH, K), jnp.int32),
            pltpu.VMEM((CH, K), jnp.int32),
            pltpu.VMEM((2, NBUF, K, CB), jnp.float32),
            pltpu.VMEM_SHARED((N, CB), jnp.float32),
            pltpu.SemaphoreType.DMA,
            pltpu.SemaphoreType.DMA,
        ],
    )
    def kern(m_h, src_h, dst_h, z_h, agg_h, sidx, didx, rows, acc_sh,
             gsem, ssem):
        c = lax.axis_index("c")
        s = lax.axis_index("s")
        pltpu.sync_copy(dst_h.at[s], didx)

        for p in range(2):
            q = 2 * p + c
            pltpu.sync_copy(src_h.at[p * NC * NS + c * NS + s], sidx)

            @pl.when(s < NS - 1)
            def _zero():
                pltpu.sync_copy(z_h, acc_sh.at[pl.ds(s * FR, FR)])

            @pl.when(s == NS - 1)
            def _zero_t():
                pltpu.sync_copy(z_h.at[pl.ds(0, TAIL)],
                                acc_sh.at[pl.ds((NS - 1) * FR, TAIL)])

            plsc.subcore_barrier()

            @pl.loop(0, CH, step=NBUF)
            def _body(j0):
                half = (j0 // NBUF) % 2

                gets = []
                for b in range(NBUF):
                    @pl.when(j0 >= 2 * NBUF)
                    def _drain_one(b=b):
                        pltpu.make_async_copy(m_h.at[sidx.at[j0]],
                                              rows.at[0, b], ssem).wait()

                    gets.append(
                        pltpu.async_copy(m_h.at[sidx.at[j0 + b]],
                                         rows.at[half, b], gsem))
                for b in range(NBUF):
                    gets[b].wait()
                    pltpu.async_copy(rows.at[half, b],
                                     acc_sh.at[didx.at[j0 + b]], ssem,
                                     add=True)

            for i in range(2 * NBUF):
                pltpu.make_async_copy(m_h.at[sidx.at[0]],
                                      rows.at[0, i % NBUF], ssem).wait()

            plsc.subcore_barrier()

            @pl.when(s < NS - 1)
            def _wb():
                pltpu.sync_copy(acc_sh.at[pl.ds(s * FR, FR)],
                                agg_h.at[pl.ds(q * N + s * FR, FR)])

            @pl.when(s == NS - 1)
            def _wb_t():
                pltpu.sync_copy(acc_sh.at[pl.ds((NS - 1) * FR, TAIL)],
                                agg_h.at[pl.ds(q * N + (NS - 1) * FR, TAIL)])

    return kern


def _sc_spmm2(mflat, srcb2, dst):
    zrows = jnp.zeros((FR, CB), jnp.float32)
    return _sc_spmm2_kern()(mflat, srcb2, dst, zrows)


@functools.cache
def _sc_spmm_l3_kern():
    @functools.partial(
        pl.kernel,
        out_type=jax.ShapeDtypeStruct((NC * N, CB), jnp.float32),
        mesh=_mesh(),
        compiler_params=pltpu.CompilerParams(use_tc_tiling_on_sc=False,
                                             skip_device_barrier=True),
        scratch_types=[
            pltpu.VMEM((CH_D, KD), jnp.int32),
            pltpu.VMEM((CH_D, KD), jnp.int32),
            pltpu.VMEM((2, NBUF, KD, CB), jnp.float32),
            pltpu.VMEM_SHARED((N, CB), jnp.float32),
            pltpu.SemaphoreType.DMA,
            pltpu.SemaphoreType.DMA,
        ],
    )
    def kern(m_h, src_h, dst_h, z_h, agg_h, sidx, didx, rows, acc_sh,
             gsem, ssem):
        c = lax.axis_index("c")
        s = lax.axis_index("s")
        t = c * NS + s
        pltpu.sync_copy(src_h.at[t], sidx)
        pltpu.sync_copy(dst_h.at[t], didx)

        @pl.when(s < NS - 1)
        def _zero():
            pltpu.sync_copy(z_h, acc_sh.at[pl.ds(s * FR, FR)])

        @pl.when(s == NS - 1)
        def _zero_t():
            pltpu.sync_copy(z_h.at[pl.ds(0, TAIL)],
                            acc_sh.at[pl.ds((NS - 1) * FR, TAIL)])

        plsc.subcore_barrier()

        @pl.loop(0, CH_D, step=NBUF)
        def _body(j0):
            half = (j0 // NBUF) % 2

            gets = []
            for b in range(NBUF):
                @pl.when(j0 >= 2 * NBUF)
                def _drain_one(b=b):
                    pltpu.make_async_copy(m_h.at[sidx.at[j0]],
                                          rows.at[0, b], ssem).wait()

                gets.append(
                    pltpu.async_copy(m_h.at[sidx.at[j0 + b]],
                                     rows.at[half, b], gsem))
            for b in range(NBUF):
                gets[b].wait()
                pltpu.async_copy(rows.at[half, b],
                                 acc_sh.at[didx.at[j0 + b]], ssem, add=True)

        for i in range(2 * NBUF):
            pltpu.make_async_copy(m_h.at[sidx.at[0]],
                                  rows.at[0, i % NBUF], ssem).wait()

        plsc.subcore_barrier()

        @pl.when(s < NS - 1)
        def _wb():
            pltpu.sync_copy(acc_sh.at[pl.ds(s * FR, FR)],
                            agg_h.at[pl.ds(c * N + s * FR, FR)])

        @pl.when(s == NS - 1)
        def _wb_t():
            pltpu.sync_copy(acc_sh.at[pl.ds((NS - 1) * FR, TAIL)],
                            agg_h.at[pl.ds(c * N + (NS - 1) * FR, TAIL)])

    return kern


def _r_of(deg_ref):
    d = deg_ref[0] + deg_ref[1]
    return lax.rsqrt(jnp.maximum(d, 1.0))


def _tc_stage1(x, W1, degp):
    def body(x_ref, w_ref, deg_ref, out_ref):
        r = _r_of(deg_ref)
        z = jnp.dot(x_ref[...], w_ref[...],
                    preferred_element_type=jnp.float32) * r
        for q in range(4):
            out_ref[q] = z[:, q * CB:(q + 1) * CB]

    return pl.pallas_call(
        body,
        grid=(N // BN,),
        in_specs=[
            pl.BlockSpec((BN, D_IN), lambda i: (i, 0)),
            pl.BlockSpec((D_IN, D_HID), lambda i: (0, 0)),
            pl.BlockSpec((NC, BN, 1), lambda i: (0, i, 0)),
        ],
        out_specs=pl.BlockSpec((4, BN, CB), lambda i: (0, i, 0)),
        out_shape=jax.ShapeDtypeStruct((4, N, CB), jnp.float32),
    )(x, W1, degp)


def _tc_stage_mid(agg, degp, b, W, nq):
    d_out = nq * CB

    def body(a_ref, deg_ref, b_ref, w_ref, out_ref):
        r = _r_of(deg_ref)
        h = jnp.concatenate(
            [a_ref[0], a_ref[1], a_ref[2], a_ref[3]], axis=1)
        h = jnp.maximum(h * r + b_ref[...], 0.0)
        z = jnp.dot(h, w_ref[...], preferred_element_type=jnp.float32) * r
        for q in range(nq):
            out_ref[q] = z[:, q * CB:(q + 1) * CB]

    return pl.pallas_call(
        body,
        grid=(N // BN,),
        in_specs=[
            pl.BlockSpec((4, BN, CB), lambda i: (0, i, 0)),
            pl.BlockSpec((NC, BN, 1), lambda i: (0, i, 0)),
            pl.BlockSpec((1, D_HID), lambda i: (0, 0)),
            pl.BlockSpec((D_HID, d_out), lambda i: (0, 0)),
        ],
        out_specs=pl.BlockSpec((nq, BN, CB), lambda i: (0, i, 0)),
        out_shape=jax.ShapeDtypeStruct((nq, N, CB), jnp.float32),
    )(agg, degp, b, W)


def _tc_stage_final(agg3, degp, b3p):
    def body(agg_ref, deg_ref, b_ref, out_ref):
        r = _r_of(deg_ref)
        o = agg_ref[0] + agg_ref[1]
        o = o * r + b_ref[...]
        col = lax.broadcasted_iota(jnp.int32, (BN, D_OUT_PAD), 1)
        valid = col < D_OUT
        om = jnp.where(valid, o, jnp.float32(-1e30))
        m = jnp.max(om, axis=1, keepdims=True)
        e = jnp.where(valid, jnp.exp(om - m), 0.0)
        lse = jnp.log(jnp.sum(e, axis=1, keepdims=True)) + m
        res = o - lse
        out_ref[...] = res[:, :D_OUT]

    return pl.pallas_call(
        body,
        grid=(N // BN,),
        in_specs=[
            pl.BlockSpec((NC, BN, CB), lambda i: (0, i, 0)),
            pl.BlockSpec((NC, BN, 1), lambda i: (0, i, 0)),
            pl.BlockSpec((1, D_OUT_PAD), lambda i: (0, 0)),
        ],
        out_specs=pl.BlockSpec((BN, D_OUT), lambda i: (i, 0)),
        out_shape=jax.ShapeDtypeStruct((N, D_OUT), jnp.float32),
    )(agg3, degp, b3p)


def kernel(x, edge_index, W1, b1, W2, b2, W3, b3):
    src = edge_index[0]
    dst = edge_index[1]

    src3 = src.reshape(NC * NS, CH_D, KD)
    dst3 = dst.reshape(NC * NS, CH_D, KD)
    dstp = dst.reshape(NS, CH, K)
    offq = (jnp.arange(4, dtype=jnp.int32) * N).reshape(2, NC, 1, 1, 1)
    srcb2 = (src.reshape(1, 1, NS, CH, K) + offq).reshape(
        2 * NC * NS, CH, K)

    W3p = jnp.pad(W3, ((0, 0), (0, D_OUT_PAD - D_OUT)))
    b3p = jnp.pad(b3, (0, D_OUT_PAD - D_OUT)).reshape(1, D_OUT_PAD)

    degp = _sc_degree(dst3).reshape(NC, N, 1)

    m1 = _tc_stage1(x, W1, degp).reshape(4 * N, CB)
    a1 = _sc_spmm2(m1, srcb2, dstp).reshape(4, N, CB)
    m2 = _tc_stage_mid(a1, degp, b1.reshape(1, D_HID), W2,
                       nq=4).reshape(4 * N, CB)
    a2 = _sc_spmm2(m2, srcb2, dstp).reshape(4, N, CB)
    m3 = _tc_stage_mid(a2, degp, b2.reshape(1, D_HID), W3p,
                       nq=1).reshape(N, CB)
    a3 = _sc_spmm_l3_kern()(m3, src3, dst3,
                            jnp.zeros((FR, CB), jnp.float32)).reshape(NC, N, CB)
    return _tc_stage_final(a3, degp, b3p)
